# baseline (speedup 1.0000x reference)
import functools

import jax
import jax.numpy as jnp
from jax import lax
from jax.experimental import pallas as pl
from jax.experimental.pallas import tpu as pltpu
from jax.experimental.pallas import tpu_sc as plsc

_NC = 2
_NS = 16
_L = 16

_SEC = 4608
_NSET = 4
_ZCH = 4096


def _build_densify(elems_per_t: int, out_features: int, in_features: int):
  n_pass = 2
  win_rows = out_features // (_NC * n_pass)
  win = win_rows * in_features
  dump = win
  stripe = win // _NS
  n_sec = elems_per_t // _SEC

  mesh = plsc.VectorSubcoreMesh(core_axis_name="c", subcore_axis_name="s")

  scratch = []
  for _ in range(_NSET):
    scratch.append(pltpu.VMEM((_SEC,), jnp.int32))
    scratch.append(pltpu.VMEM((_SEC,), jnp.float32))
  scratch += [
      pltpu.VMEM((_ZCH,), jnp.float32),
      pltpu.VMEM_SHARED((win + 8,), jnp.float32),
      pltpu.SemaphoreType.DMA,
      pltpu.SemaphoreType.DMA,
      pltpu.SemaphoreType.DMA,
  ]

  @functools.partial(
      pl.kernel,
      mesh=mesh,
      out_type=jax.ShapeDtypeStruct((out_features * in_features,),
                                    jnp.float32),
      scratch_types=scratch,
  )
  def densify(flat_hbm, vals_hbm, w_hbm, *bufs):
    sets = [(bufs[2 * q], bufs[2 * q + 1]) for q in range(_NSET)]
    zbuf, acc, sem, insem, zsem = bufs[2 * _NSET:]
    c = lax.axis_index("c")
    s = lax.axis_index("s")

    def _fire_in(sec, q):
      fb, vb = sets[q]
      pltpu.async_copy(flat_hbm.at[s, pl.ds(sec * _SEC, _SEC)], fb, insem)
      pltpu.async_copy(vals_hbm.at[s, pl.ds(sec * _SEC, _SEC)], vb, insem)

    def _wait_in(q):
      fb, vb = sets[q]
      pltpu.make_async_copy(flat_hbm.at[s, pl.ds(0, _SEC)], fb,
                            insem).wait()
      pltpu.make_async_copy(vals_hbm.at[s, pl.ds(0, _SEC)], vb,
                            insem).wait()

    def _fire_scatter(q):
      fb, vb = sets[q]
      pltpu.async_copy(vb, acc.at[fb], sem, add=True)

    def _drain_scatter(q):
      fb, vb = sets[q]
      pltpu.make_async_copy(vb, acc.at[fb], sem).wait()

    @pl.loop(0, _ZCH // _L)
    def _z(i):
      zbuf[pl.ds(i * _L, _L)] = jnp.zeros((_L,), jnp.float32)

    n_z = stripe // _ZCH
    my0 = s * stripe

    def _fire_zero():
      @pl.loop(0, n_z)
      def _zf(i):
        pltpu.async_copy(zbuf, acc.at[pl.ds(my0 + i * _ZCH, _ZCH)], zsem)

    def _drain_zero():
      @pl.loop(0, n_z)
      def _zd(i):
        pltpu.make_async_copy(
            zbuf, acc.at[pl.ds(my0 + i * _ZCH, _ZCH)], zsem).wait()

    _fire_zero()
    _drain_zero()
    plsc.subcore_barrier()

    for p in range(n_pass):
      row_base = (c * n_pass + p) * win_rows
      base = row_base * in_features

      _fire_in(0, 0)
      _fire_in(1, 1)

      @pl.loop(0, n_sec // _NSET)
      def _rnd(h, base=base):
        for q in range(_NSET):
          sec = h * _NSET + q
          _wait_in(q)
          fb, _ = sets[q]

          @pl.loop(0, _SEC // _L, unroll=8)
          def _grp(i, fb=fb, base=base):
            sl = pl.ds(i * _L, _L)
            local = fb[sl] - base
            ok = local.astype(jnp.uint32) < jnp.uint32(win)
            fb[sl] = jnp.where(ok, local, dump)

          _fire_scatter(q)
          qn = (q + 2) % _NSET

          @pl.when((sec >= 2) & (sec + 2 < n_sec))
          def _(qn=qn):
            _drain_scatter(qn)

          @pl.when(sec + 2 < n_sec)
          def _(sec=sec, qn=qn):
            _fire_in(sec + 2, qn)

      for q in range(_NSET):
        _drain_scatter(q)
      plsc.subcore_barrier()

      pltpu.sync_copy(acc.at[pl.ds(my0, stripe)],
                      w_hbm.at[pl.ds(base + my0, stripe)])

      if p + 1 < n_pass:
        _fire_zero()
        _drain_zero()
        plsc.subcore_barrier()

  return densify


def _mm_body(x_ref, w_ref, b_ref, o_ref):
  acc = lax.dot_general(x_ref[...], w_ref[...], (((1,), (1,)), ((), ())))
  o_ref[...] = acc + b_ref[...]


def _matmul(x, w2d, bias2d, bm: int, bn: int):
  batch, in_features = x.shape
  out_features = bias2d.shape[1]
  grid = (batch // bm, out_features // bn)
  return pl.pallas_call(
      _mm_body,
      grid=grid,
      in_specs=[
          pl.BlockSpec((bm, in_features), lambda i, j: (i, 0)),
          pl.BlockSpec((bn, in_features), lambda i, j: (j, 0)),
          pl.BlockSpec((1, bn), lambda i, j: (0, j)),
      ],
      out_specs=pl.BlockSpec((bm, bn), lambda i, j: (i, j)),
      out_shape=jax.ShapeDtypeStruct((batch, out_features), jnp.float32),
  )(x, w2d, bias2d)


def kernel(inputs, indices, active_fwd_weights, bias):
  batch, in_features = inputs.shape
  out_features = bias.shape[0]
  nnz = indices.shape[1]

  chunk = _NSET * _SEC
  elems_per_t = -(-nnz // (_NS * chunk)) * chunk
  padded = _NS * elems_per_t
  pad = padded - nnz

  flat = indices[0] * in_features + indices[1]
  flat = jnp.concatenate(
      [flat, jnp.full((pad,), out_features * in_features, jnp.int32)])
  vals = jnp.concatenate(
      [active_fwd_weights, jnp.zeros((pad,), jnp.float32)])
  flat2 = flat.reshape(_NS, elems_per_t)
  vals2 = vals.reshape(_NS, elems_per_t)

  densify = _build_densify(elems_per_t, out_features, in_features)
  w_flat = densify(flat2, vals2)
  w2d = w_flat.reshape(out_features, in_features)

  return _matmul(inputs, w2d, bias.reshape(1, out_features), bm=512, bn=512)
